# baseline (speedup 1.0000x reference)
import functools

import jax
import jax.numpy as jnp
from jax import lax
from jax.experimental import pallas as pl
from jax.experimental.pallas import tpu as pltpu
from jax.experimental.pallas import tpu_sc as plsc

_L = 16


def kernel(x, f_sonn, x0, dx):
    nt, n1, n2 = f_sonn.shape
    flat_tab = f_sonn.reshape(nt * n1 * n2)

    mesh = plsc.VectorSubcoreMesh(core_axis_name="c", subcore_axis_name="s")

    @functools.partial(
        pl.kernel,
        mesh=mesh,
        out_type=jax.ShapeDtypeStruct((_L,), jnp.float32),
        scratch_types=[
            pltpu.VMEM((_L,), jnp.float32),
            pltpu.VMEM((_L,), jnp.float32),
            pltpu.VMEM((_L,), jnp.float32),
            pltpu.VMEM((_L,), jnp.int32),
            pltpu.VMEM((_L,), jnp.float32),
            pltpu.SemaphoreType.DMA,
        ],
    )
    def _lookup(x_hbm, x0_hbm, dx_hbm, tab_hbm, out_hbm,
                xs, x0s, dxs, idxv, valv, sem):
        cid = lax.axis_index("c")
        sid = lax.axis_index("s")

        @pl.when(jnp.logical_and(cid == 0, sid == 0))
        def _():
            dxs[...] = jnp.full((_L,), 1.0, jnp.float32)
            pltpu.sync_copy(x_hbm, xs.at[pl.ds(0, 3)])
            pltpu.sync_copy(x0_hbm, x0s.at[pl.ds(0, 3)])
            pltpu.sync_copy(dx_hbm, dxs.at[pl.ds(0, 3)])
            qv = (xs[...] - x0s[...]) / dxs[...]

            def comp(i, dim):
                q = qv[i]
                r = q.astype(jnp.int32)
                ici = jnp.where(r.astype(jnp.float32) > q, r - 1, r)
                return jnp.minimum(jnp.maximum(ici, 0), dim - 1)

            flat = (comp(0, nt) * (n1 * n2)
                    + comp(1, n1) * n2
                    + comp(2, n2))
            idxv[...] = jnp.full((_L,), flat, jnp.int32)
            pltpu.async_copy(tab_hbm.at[idxv], valv, sem).wait()
            pltpu.sync_copy(valv, out_hbm)

    res = _lookup(x, x0, dx, flat_tab)
    return res[0]


# speedup vs baseline: 9.6012x; 9.6012x over previous
import functools

import jax
import jax.numpy as jnp
from jax import lax
from jax.experimental import pallas as pl
from jax.experimental.pallas import tpu as pltpu
from jax.experimental.pallas import tpu_sc as plsc

_L = 16


def kernel(x, f_sonn, x0, dx):
    nt, n1, n2 = f_sonn.shape

    mesh = plsc.VectorSubcoreMesh(core_axis_name="c", subcore_axis_name="s")

    @functools.partial(
        pl.kernel,
        mesh=mesh,
        out_type=jax.ShapeDtypeStruct((_L,), jnp.float32),
        scratch_types=[
            pltpu.VMEM((_L,), jnp.float32),
            pltpu.VMEM((_L,), jnp.float32),
            pltpu.VMEM((_L,), jnp.float32),
            pltpu.VMEM((8, 128), jnp.float32),
            pltpu.VMEM((_L,), jnp.float32),
        ],
    )
    def _lookup(x_hbm, x0_hbm, dx_hbm, tab_hbm, out_hbm,
                xs, x0s, dxs, tilev, valv):
        cid = lax.axis_index("c")
        sid = lax.axis_index("s")

        @pl.when(jnp.logical_and(cid == 0, sid == 0))
        def _():
            dxs[...] = jnp.full((_L,), 1.0, jnp.float32)
            pltpu.sync_copy(x_hbm, xs.at[pl.ds(0, 3)])
            pltpu.sync_copy(x0_hbm, x0s.at[pl.ds(0, 3)])
            pltpu.sync_copy(dx_hbm, dxs.at[pl.ds(0, 3)])
            qv = (xs[...] - x0s[...]) / dxs[...]

            def comp(i, dim):
                q = qv[i]
                r = q.astype(jnp.int32)
                ici = jnp.where(r.astype(jnp.float32) > q, r - 1, r)
                return jnp.minimum(jnp.maximum(ici, 0), dim - 1)

            i0 = comp(0, nt)
            i1 = comp(1, n1)
            i2 = comp(2, n2)
            sub = jnp.bitwise_and(i1, 7)
            lane = jnp.bitwise_and(i2, 127)
            row0 = pl.multiple_of(i1 - sub, 8)
            col0 = pl.multiple_of(i2 - lane, 128)
            pltpu.sync_copy(
                tab_hbm.at[i0, pl.ds(row0, 8), pl.ds(col0, 128)],
                tilev)
            lane16 = jnp.bitwise_and(lane, 127 - 15)
            rem = lane - lane16
            v = tilev[sub, pl.ds(lane16, _L)]
            val = v[0]
            for j in range(1, _L):
                val = jnp.where(rem == j, v[j], val)
            valv[...] = jnp.full((_L,), val, jnp.float32)
            pltpu.sync_copy(valv, out_hbm)

    res = _lookup(x, x0, dx, f_sonn)
    return res[0]


# speedup vs baseline: 10.3565x; 1.0787x over previous
import functools

import jax
import jax.numpy as jnp
from jax import lax
from jax.experimental import pallas as pl
from jax.experimental.pallas import tpu as pltpu
from jax.experimental.pallas import tpu_sc as plsc

_L = 16


def kernel(x, f_sonn, x0, dx):
    nt, n1, n2 = f_sonn.shape

    mesh = plsc.VectorSubcoreMesh(
        core_axis_name="c", subcore_axis_name="s", num_cores=1)

    @functools.partial(
        pl.kernel,
        mesh=mesh,
        out_type=jax.ShapeDtypeStruct((_L,), jnp.float32),
        scratch_types=[
            pltpu.VMEM((_L,), jnp.float32),
            pltpu.VMEM((_L,), jnp.float32),
            pltpu.VMEM((_L,), jnp.float32),
            pltpu.VMEM((8, 128), jnp.float32),
            pltpu.VMEM((_L,), jnp.float32),
        ],
    )
    def _lookup(x_hbm, x0_hbm, dx_hbm, tab_hbm, out_hbm,
                xs, x0s, dxs, tilev, valv):
        cid = lax.axis_index("c")
        sid = lax.axis_index("s")

        @pl.when(jnp.logical_and(cid == 0, sid == 0))
        def _():
            dxs[...] = jnp.full((_L,), 1.0, jnp.float32)
            pltpu.sync_copy(x_hbm, xs.at[pl.ds(0, 3)])
            pltpu.sync_copy(x0_hbm, x0s.at[pl.ds(0, 3)])
            pltpu.sync_copy(dx_hbm, dxs.at[pl.ds(0, 3)])
            qv = (xs[...] - x0s[...]) / dxs[...]

            def comp(i, dim):
                q = qv[i]
                r = q.astype(jnp.int32)
                ici = jnp.where(r.astype(jnp.float32) > q, r - 1, r)
                return jnp.minimum(jnp.maximum(ici, 0), dim - 1)

            i0 = comp(0, nt)
            i1 = comp(1, n1)
            i2 = comp(2, n2)
            sub = jnp.bitwise_and(i1, 7)
            lane = jnp.bitwise_and(i2, 127)
            row0 = pl.multiple_of(i1 - sub, 8)
            col0 = pl.multiple_of(i2 - lane, 128)
            pltpu.sync_copy(
                tab_hbm.at[i0, pl.ds(row0, 8), pl.ds(col0, 128)],
                tilev)
            lane16 = jnp.bitwise_and(lane, 127 - 15)
            rem = lane - lane16
            v = tilev[sub, pl.ds(lane16, _L)]
            val = v[0]
            for j in range(1, _L):
                val = jnp.where(rem == j, v[j], val)
            valv[...] = jnp.full((_L,), val, jnp.float32)
            pltpu.sync_copy(valv, out_hbm)

    res = _lookup(x, x0, dx, f_sonn)
    return res[0]


# speedup vs baseline: 10.8229x; 1.0450x over previous
import functools

import jax
import jax.numpy as jnp
from jax import lax
from jax.experimental import pallas as pl
from jax.experimental.pallas import tpu as pltpu
from jax.experimental.pallas import tpu_sc as plsc

_L = 16


def kernel(x, f_sonn, x0, dx):
    nt, n1, n2 = f_sonn.shape

    mesh = plsc.VectorSubcoreMesh(
        core_axis_name="c", subcore_axis_name="s", num_cores=1)

    @functools.partial(
        pl.kernel,
        mesh=mesh,
        out_type=jax.ShapeDtypeStruct((_L,), jnp.float32),
        scratch_types=[
            pltpu.VMEM((_L,), jnp.float32),
            pltpu.VMEM((_L,), jnp.float32),
            pltpu.VMEM((_L,), jnp.float32),
            pltpu.VMEM((8, 128), jnp.float32),
            pltpu.VMEM((_L,), jnp.float32),
            pltpu.SemaphoreType.DMA,
            pltpu.SemaphoreType.DMA,
            pltpu.SemaphoreType.DMA,
        ],
    )
    def _lookup(x_hbm, x0_hbm, dx_hbm, tab_hbm, out_hbm,
                xs, x0s, dxs, tilev, valv, s1, s2, s3):
        cid = lax.axis_index("c")
        sid = lax.axis_index("s")

        @pl.when(jnp.logical_and(cid == 0, sid == 0))
        def _():
            c1 = pltpu.async_copy(x_hbm, xs.at[pl.ds(0, 3)], s1)
            c2 = pltpu.async_copy(x0_hbm, x0s.at[pl.ds(0, 3)], s2)
            c3 = pltpu.async_copy(dx_hbm, dxs.at[pl.ds(0, 3)], s3)
            c1.wait()
            c2.wait()
            c3.wait()
            qv = (xs[...] - x0s[...]) / dxs[...]

            def comp(i, dim):
                q = qv[i]
                r = q.astype(jnp.int32)
                ici = jnp.where(r.astype(jnp.float32) > q, r - 1, r)
                return jnp.minimum(jnp.maximum(ici, 0), dim - 1)

            i0 = comp(0, nt)
            i1 = comp(1, n1)
            i2 = comp(2, n2)
            sub = jnp.bitwise_and(i1, 7)
            lane = jnp.bitwise_and(i2, 127)
            row0 = pl.multiple_of(i1 - sub, 8)
            col0 = pl.multiple_of(i2 - lane, 128)
            pltpu.sync_copy(
                tab_hbm.at[i0, pl.ds(row0, 8), pl.ds(col0, 128)],
                tilev)
            lane16 = jnp.bitwise_and(lane, 127 - 15)
            rem = lane - lane16
            v = tilev[sub, pl.ds(lane16, _L)]
            val = v[0]
            for j in range(1, _L):
                val = jnp.where(rem == j, v[j], val)
            valv[...] = jnp.full((_L,), val, jnp.float32)
            pltpu.sync_copy(valv, out_hbm)

    res = _lookup(x, x0, dx, f_sonn)
    return res[0]


# speedup vs baseline: 10.8687x; 1.0042x over previous
import functools

import jax
import jax.numpy as jnp
from jax import lax
from jax.experimental import pallas as pl
from jax.experimental.pallas import tpu as pltpu
from jax.experimental.pallas import tpu_sc as plsc

_L = 16


def kernel(x, f_sonn, x0, dx):
    nt, n1, n2 = f_sonn.shape

    mesh = plsc.VectorSubcoreMesh(
        core_axis_name="c", subcore_axis_name="s",
        num_cores=1, num_subcores=1)

    @functools.partial(
        pl.kernel,
        mesh=mesh,
        out_type=jax.ShapeDtypeStruct((_L,), jnp.float32),
        scratch_types=[
            pltpu.VMEM((_L,), jnp.float32),
            pltpu.VMEM((_L,), jnp.float32),
            pltpu.VMEM((_L,), jnp.float32),
            pltpu.VMEM((8, 128), jnp.float32),
            pltpu.VMEM((_L,), jnp.float32),
            pltpu.SemaphoreType.DMA,
            pltpu.SemaphoreType.DMA,
            pltpu.SemaphoreType.DMA,
        ],
    )
    def _lookup(x_hbm, x0_hbm, dx_hbm, tab_hbm, out_hbm,
                xs, x0s, dxs, tilev, valv, s1, s2, s3):
        cid = lax.axis_index("c")
        sid = lax.axis_index("s")

        @pl.when(jnp.logical_and(cid == 0, sid == 0))
        def _():
            c1 = pltpu.async_copy(x_hbm, xs.at[pl.ds(0, 3)], s1)
            c2 = pltpu.async_copy(x0_hbm, x0s.at[pl.ds(0, 3)], s2)
            c3 = pltpu.async_copy(dx_hbm, dxs.at[pl.ds(0, 3)], s3)
            c1.wait()
            c2.wait()
            c3.wait()
            qv = (xs[...] - x0s[...]) / dxs[...]

            def comp(i, dim):
                q = qv[i]
                r = q.astype(jnp.int32)
                ici = jnp.where(r.astype(jnp.float32) > q, r - 1, r)
                return jnp.minimum(jnp.maximum(ici, 0), dim - 1)

            i0 = comp(0, nt)
            i1 = comp(1, n1)
            i2 = comp(2, n2)
            sub = jnp.bitwise_and(i1, 7)
            lane = jnp.bitwise_and(i2, 127)
            row0 = pl.multiple_of(i1 - sub, 8)
            col0 = pl.multiple_of(i2 - lane, 128)
            pltpu.sync_copy(
                tab_hbm.at[i0, pl.ds(row0, 8), pl.ds(col0, 128)],
                tilev)
            lane16 = jnp.bitwise_and(lane, 127 - 15)
            rem = lane - lane16
            v = tilev[sub, pl.ds(lane16, _L)]
            val = v[0]
            for j in range(1, _L):
                val = jnp.where(rem == j, v[j], val)
            valv[...] = jnp.full((_L,), val, jnp.float32)
            pltpu.sync_copy(valv, out_hbm)

    res = _lookup(x, x0, dx, f_sonn)
    return res[0]


# speedup vs baseline: 10.9044x; 1.0033x over previous
import functools

import jax
import jax.numpy as jnp
from jax import lax
from jax.experimental import pallas as pl
from jax.experimental.pallas import tpu as pltpu
from jax.experimental.pallas import tpu_sc as plsc

_L = 16


def kernel(x, f_sonn, x0, dx):
    nt, n1, n2 = f_sonn.shape

    mesh = plsc.VectorSubcoreMesh(
        core_axis_name="c", subcore_axis_name="s",
        num_cores=1, num_subcores=1)

    @functools.partial(
        pl.kernel,
        mesh=mesh,
        out_type=jax.ShapeDtypeStruct((_L,), jnp.float32),
        scratch_types=[
            pltpu.VMEM((_L,), jnp.float32),
            pltpu.VMEM((_L,), jnp.float32),
            pltpu.VMEM((_L,), jnp.float32),
            pltpu.VMEM((8, 128), jnp.float32),
            pltpu.VMEM((_L,), jnp.float32),
            pltpu.SemaphoreType.DMA,
            pltpu.SemaphoreType.DMA,
            pltpu.SemaphoreType.DMA,
        ],
    )
    def _lookup(x_hbm, x0_hbm, dx_hbm, tab_hbm, out_hbm,
                xs, x0s, dxs, tilev, valv, s1, s2, s3):
        c1 = pltpu.async_copy(x_hbm, xs.at[pl.ds(0, 3)], s1)
        c2 = pltpu.async_copy(x0_hbm, x0s.at[pl.ds(0, 3)], s2)
        c3 = pltpu.async_copy(dx_hbm, dxs.at[pl.ds(0, 3)], s3)
        c1.wait()
        c2.wait()
        c3.wait()
        qv = (xs[...] - x0s[...]) / dxs[...]
        rv = qv.astype(jnp.int32)
        icv = jnp.where(rv.astype(jnp.float32) > qv, rv - 1, rv)
        lanes = lax.iota(jnp.int32, _L)
        dimv = jnp.where(lanes == 0, nt - 1,
                         jnp.where(lanes == 1, n1 - 1, n2 - 1))
        icv = jnp.minimum(jnp.maximum(icv, 0), dimv)
        i0 = icv[0]
        i1 = icv[1]
        i2 = icv[2]
        sub = jnp.bitwise_and(i1, 7)
        lane = jnp.bitwise_and(i2, 127)
        row0 = pl.multiple_of(i1 - sub, 8)
        col0 = pl.multiple_of(i2 - lane, 128)
        pltpu.sync_copy(
            tab_hbm.at[i0, pl.ds(row0, 8), pl.ds(col0, 128)],
            tilev)
        lane16 = jnp.bitwise_and(lane, 127 - 15)
        rem = lane - lane16
        v = tilev[sub, pl.ds(lane16, _L)]
        valv[...] = lax.gather(
            v, jnp.full((_L,), rem, jnp.int32)[:, None],
            lax.GatherDimensionNumbers(
                offset_dims=(), collapsed_slice_dims=(0,),
                start_index_map=(0,)),
            (1,), mode=lax.GatherScatterMode.PROMISE_IN_BOUNDS)
        pltpu.sync_copy(valv, out_hbm)

    res = _lookup(x, x0, dx, f_sonn)
    return res[0]


# speedup vs baseline: 10.9120x; 1.0007x over previous
import functools

import jax
import jax.numpy as jnp
from jax import lax
from jax.experimental import pallas as pl
from jax.experimental.pallas import tpu as pltpu
from jax.experimental.pallas import tpu_sc as plsc

_L = 16


def kernel(x, f_sonn, x0, dx):
    nt, n1, n2 = f_sonn.shape

    mesh = plsc.VectorSubcoreMesh(
        core_axis_name="c", subcore_axis_name="s",
        num_cores=1, num_subcores=1)

    dnums = lax.GatherDimensionNumbers(
        offset_dims=(), collapsed_slice_dims=(0,), start_index_map=(0,))

    @functools.partial(
        pl.kernel,
        mesh=mesh,
        out_type=jax.ShapeDtypeStruct((_L,), jnp.float32),
        scratch_types=[
            pltpu.VMEM((_L,), jnp.float32),
            pltpu.VMEM((8, 128), jnp.float32),
            pltpu.VMEM((_L,), jnp.float32),
        ],
    )
    def _lookup(xin_hbm, tab_hbm, out_hbm, xs, tilev, valv):
        pltpu.sync_copy(xin_hbm, xs.at[pl.ds(0, 9)])
        vin = xs[...]
        lanes = lax.iota(jnp.int32, _L)

        def shift(vec, off):
            idx = jnp.bitwise_and(lanes + off, _L - 1)
            return lax.gather(vec, idx[:, None], dnums, (1,),
                              mode=lax.GatherScatterMode.PROMISE_IN_BOUNDS)

        qv = (vin - shift(vin, 3)) / shift(vin, 6)
        rv = qv.astype(jnp.int32)
        icv = jnp.where(rv.astype(jnp.float32) > qv, rv - 1, rv)
        dimv = jnp.where(lanes == 0, nt - 1,
                         jnp.where(lanes == 1, n1 - 1, n2 - 1))
        icv = jnp.minimum(jnp.maximum(icv, 0), dimv)
        i0 = icv[0]
        i1 = icv[1]
        i2 = icv[2]
        sub = jnp.bitwise_and(i1, 7)
        lane = jnp.bitwise_and(i2, 127)
        row0 = pl.multiple_of(i1 - sub, 8)
        col0 = pl.multiple_of(i2 - lane, 128)
        pltpu.sync_copy(
            tab_hbm.at[i0, pl.ds(row0, 8), pl.ds(col0, 128)],
            tilev)
        lane16 = jnp.bitwise_and(lane, 127 - 15)
        rem = lane - lane16
        v = tilev[sub, pl.ds(lane16, _L)]
        valv[...] = lax.gather(
            v, jnp.full((_L,), rem, jnp.int32)[:, None], dnums,
            (1,), mode=lax.GatherScatterMode.PROMISE_IN_BOUNDS)
        pltpu.sync_copy(valv, out_hbm)

    res = _lookup(jnp.concatenate([x, x0, dx]), f_sonn)
    return res[0]
